# Initial kernel scaffold; baseline (speedup 1.0000x reference)
import jax
import jax.numpy as jnp
from jax.experimental import pallas as pl


def kernel(pos, embed_x, embed_y):
    raise NotImplementedError("write your pallas kernel here")


# speedup vs baseline: 4.1335x; 4.1335x over previous
import functools

import jax
import jax.numpy as jnp
from jax import lax
from jax.experimental import pallas as pl
from jax.experimental.pallas import tpu as pltpu
from jax.experimental.pallas import tpu_sc as plsc

GRID_N = 1000
D = 64
NC = 2
NS = 16
NW = NC * NS
GB = 128
K = 8
CHUNK = K * GB


@functools.partial(jax.jit, static_argnums=(1,))
def _lookup(args, total):
    idx_flat, table = args
    n_per_w = total // NW
    n_blocks = n_per_w // CHUNK
    mesh = plsc.VectorSubcoreMesh(core_axis_name="c", subcore_axis_name="s")

    @functools.partial(
        pl.kernel,
        out_type=jax.ShapeDtypeStruct((total, D), jnp.float32),
        mesh=mesh,
        scratch_types=[
            pltpu.VMEM((CHUNK,), jnp.int32),
            pltpu.VMEM((CHUNK, D), jnp.float32),
            pltpu.SemaphoreType.DMA,
        ],
        compiler_params=pltpu.CompilerParams(use_tc_tiling_on_sc=False),
    )
    def emb(idx_hbm, table_hbm, out_hbm, idx_v, rows_v, sem):
        wid = lax.axis_index("s") * NC + lax.axis_index("c")
        woff = wid * n_per_w
        offs = lax.rem(lax.iota(jnp.int32, 16), 2) * GRID_N

        def block(b, carry):
            base = woff + b * CHUNK
            pltpu.sync_copy(idx_hbm.at[pl.ds(base, CHUNK)], idx_v)

            def addoff(i, c):
                sl = pl.ds(i * 16, 16)
                idx_v[sl] = idx_v[sl] + offs
                return c

            lax.fori_loop(0, CHUNK // 16, addoff, 0)

            copies = [
                pltpu.async_copy(
                    table_hbm.at[idx_v.at[pl.ds(j * GB, GB)]],
                    rows_v.at[pl.ds(j * GB, GB)],
                    sem,
                )
                for j in range(K)
            ]
            for c in copies:
                c.wait()
            pltpu.sync_copy(rows_v, out_hbm.at[pl.ds(base, CHUNK)])
            return carry

        lax.fori_loop(0, n_blocks, block, 0)

    return emb(idx_flat, table)


def kernel(pos, embed_x, embed_y):
    B, L, _ = pos.shape
    total = B * L * 2
    idx_flat = pos.reshape(total).astype(jnp.int32)
    table = jnp.concatenate((embed_x, embed_y), axis=0)
    out = _lookup((idx_flat, table), total)
    return out.reshape(B, L, 2 * D)


# speedup vs baseline: 33.4890x; 8.1018x over previous
import functools

import jax
import jax.numpy as jnp
from jax import lax
from jax.experimental import pallas as pl
from jax.experimental.pallas import tpu as pltpu
from jax.experimental.pallas import tpu_sc as plsc

D = 64
NC = 2
NS = 16
NW = NC * NS
GB = 128
KP = 2


@functools.partial(jax.jit, static_argnums=(1, 2))
def _lookup(args, B, L):
    idx_runs, ex3, ey3 = args
    n_pairs = idx_runs.shape[0] // 2
    nb_blocks = B // GB
    pairs_per_w = n_pairs // NW
    n_steps = pairs_per_w // KP
    assert n_steps % 2 == 0 and n_steps >= 4
    mesh = plsc.VectorSubcoreMesh(core_axis_name="c", subcore_axis_name="s")

    @functools.partial(
        pl.kernel,
        out_type=jax.ShapeDtypeStruct((B, L, 2 * D), jnp.float32),
        mesh=mesh,
        scratch_types=[
            pltpu.VMEM_SHARED((1000, 1, D), jnp.float32),
            pltpu.VMEM_SHARED((1000, 1, D), jnp.float32),
            [pltpu.VMEM((2 * KP, GB), jnp.int32)] * 2,
            [pltpu.VMEM((KP, GB, 1, D), jnp.float32)] * 2,
            [pltpu.VMEM((KP, GB, 1, D), jnp.float32)] * 2,
            [pltpu.SemaphoreType.DMA] * 2,
            [pltpu.SemaphoreType.DMA] * 2,
            [pltpu.SemaphoreType.DMA] * 2,
        ],
        compiler_params=pltpu.CompilerParams(use_tc_tiling_on_sc=False),
    )
    def emb(idx_hbm, ex_hbm, ey_hbm, out_hbm, ex_sh, ey_sh,
            idx_v, bx, by, isem, gsem, ssem):
        wid = lax.axis_index("s") * NC + lax.axis_index("c")
        pair0 = wid * pairs_per_w

        @pl.when(lax.axis_index("s") == 0)
        def _():
            pltpu.sync_copy(ex_hbm, ex_sh)
            pltpu.sync_copy(ey_hbm, ey_sh)

        plsc.subcore_barrier()

        def idx_src(step):
            return idx_hbm.at[pl.ds((pair0 + step * KP) * 2, 2 * KP)]

        def stage(step, p):
            pltpu.async_copy(idx_src(step), idx_v[p], isem[p])

        def fire_gathers(step, p):
            pltpu.make_async_copy(idx_src(step), idx_v[p], isem[p]).wait()
            for k in range(KP):
                pltpu.async_copy(
                    ex_sh.at[idx_v[p].at[2 * k]], bx[p].at[k], gsem[p])
                pltpu.async_copy(
                    ey_sh.at[idx_v[p].at[2 * k + 1]], by[p].at[k], gsem[p])

        def out_dst(step, k, c):
            pid = pair0 + step * KP + k
            l = pid // nb_blocks
            tb = pid - l * nb_blocks
            return out_hbm.at[pl.ds(tb * GB, GB), pl.ds(l, 1), pl.ds(c * D, D)]

        def drain_stores(step, p):
            for k in range(KP):
                pltpu.make_async_copy(bx[p].at[k], out_dst(step, k, 0), ssem[p]).wait()
                pltpu.make_async_copy(by[p].at[k], out_dst(step, k, 1), ssem[p]).wait()

        def finish(step, p):
            for k in range(KP):
                pltpu.make_async_copy(
                    ex_sh.at[idx_v[p].at[2 * k]], bx[p].at[k], gsem[p]).wait()
                pltpu.make_async_copy(
                    ey_sh.at[idx_v[p].at[2 * k + 1]], by[p].at[k], gsem[p]).wait()
            for k in range(KP):
                pltpu.async_copy(bx[p].at[k], out_dst(step, k, 0), ssem[p])
                pltpu.async_copy(by[p].at[k], out_dst(step, k, 1), ssem[p])

        stage(0, 0)
        fire_gathers(0, 0)
        stage(1, 1)

        def body(s2, carry):
            for q in (0, 1):
                s = s2 * 2 + q
                ns = s + 1
                nns = s + 2

                @pl.when(ns < n_steps)
                def _():
                    @pl.when(ns >= 2)
                    def _():
                        drain_stores(ns - 2, 1 - q)
                    fire_gathers(ns, 1 - q)

                finish(s, q)

                @pl.when(nns < n_steps)
                def _():
                    stage(nns, q)
            return carry

        lax.fori_loop(0, n_steps // 2, body, 0)
        drain_stores(n_steps - 2, 0)
        drain_stores(n_steps - 1, 1)

    return emb(idx_runs, ex3, ey3)


def kernel(pos, embed_x, embed_y):
    B, L, _ = pos.shape
    idx_runs = (
        pos.astype(jnp.int32)
        .reshape(B // GB, GB, L, 2)
        .transpose(2, 0, 3, 1)
        .reshape((L * (B // GB)) * 2, GB)
    )
    ex3, ey3 = lax.optimization_barrier(
        (embed_x.reshape(1000, 1, D), embed_y.reshape(1000, 1, D)))
    out = _lookup((idx_runs, ex3, ey3), B, L)
    return out
